# baseline (device time: 133227 ns/iter reference)
import jax
import jax.numpy as jnp
from jax import lax
from jax.experimental import pallas as pl
from jax.experimental.pallas import tpu as pltpu

K = 8
LC = 8


def kernel(x):
    M, N = x.shape
    NCOL = N // 2
    H = M // 2
    M_OUT = 2 * M
    CH = H // K
    LCH = M // LC

    def body(x_ref, out_ref, vsend, vrecv1, vrecv2, vstage,
             stage_sems, p1_send, p1_recv, p2_send, p2_recv,
             st1_sems, st2_sems, lc_in_sems, lc_out_sems):
        mx = lax.axis_index("x")
        my = lax.axis_index("y")

        barrier = pltpu.get_barrier_semaphore()
        pl.semaphore_signal(barrier, inc=1, device_id=(1 - mx, my),
                            device_id_type=pl.DeviceIdType.MESH)
        pl.semaphore_signal(barrier, inc=1, device_id=(mx, 1 - my),
                            device_id_type=pl.DeviceIdType.MESH)
        pl.semaphore_wait(barrier, 2)

        stage = []
        for k in range(K):
            c = pltpu.make_async_copy(
                x_ref.at[pl.ds(my * H + k * CH, CH),
                         pl.ds((1 - mx) * NCOL, NCOL)],
                vsend.at[pl.ds(k * CH, CH), :],
                stage_sems.at[k],
            )
            c.start()
            stage.append(c)
        p1 = []
        for k in range(K):
            stage[k].wait()
            r = pltpu.make_async_remote_copy(
                src_ref=vsend.at[pl.ds(k * CH, CH), :],
                dst_ref=vrecv1.at[pl.ds(k * CH, CH), :],
                send_sem=p1_send.at[k],
                recv_sem=p1_recv.at[k],
                device_id=(1 - mx, my),
                device_id_type=pl.DeviceIdType.MESH,
            )
            r.start()
            p1.append(r)

        lc_in = []
        for k in range(LC):
            c = pltpu.make_async_copy(
                x_ref.at[pl.ds(k * LCH, LCH), pl.ds(mx * NCOL, NCOL)],
                vstage.at[k],
                lc_in_sems.at[k],
            )
            c.start()
            lc_in.append(c)

        p2 = []
        st1 = []
        for k in range(K):
            p1[k].wait_recv()
            r = pltpu.make_async_remote_copy(
                src_ref=vrecv1.at[pl.ds(k * CH, CH), :],
                dst_ref=vrecv2.at[pl.ds(k * CH, CH), :],
                send_sem=p2_send.at[k],
                recv_sem=p2_recv.at[k],
                device_id=(mx, 1 - my),
                device_id_type=pl.DeviceIdType.MESH,
            )
            r.start()
            p2.append(r)
            s = pltpu.make_async_copy(
                vrecv1.at[pl.ds(k * CH, CH), :],
                out_ref.at[pl.ds((1 - mx) * M + my * H + k * CH, CH), :],
                st1_sems.at[k],
            )
            s.start()
            st1.append(s)

        lc_out = []
        for k in range(LC):
            lc_in[k].wait()
            c = pltpu.make_async_copy(
                vstage.at[k],
                out_ref.at[pl.ds(mx * M + k * LCH, LCH), :],
                lc_out_sems.at[k],
            )
            c.start()
            lc_out.append(c)

        st2 = []
        for k in range(K):
            p2[k].wait_recv()
            s = pltpu.make_async_copy(
                vrecv2.at[pl.ds(k * CH, CH), :],
                out_ref.at[pl.ds((1 - mx) * M + (1 - my) * H + k * CH, CH), :],
                st2_sems.at[k],
            )
            s.start()
            st2.append(s)

        for k in range(K):
            p1[k].wait_send()
            p2[k].wait_send()
            st1[k].wait()
            st2[k].wait()
        for k in range(LC):
            lc_out[k].wait()

    return pl.pallas_call(
        body,
        out_shape=jax.ShapeDtypeStruct((M_OUT, NCOL), jnp.float32),
        in_specs=[pl.BlockSpec(memory_space=pl.ANY)],
        out_specs=pl.BlockSpec(memory_space=pl.ANY),
        scratch_shapes=[
            pltpu.VMEM((H, NCOL), jnp.float32),
            pltpu.VMEM((H, NCOL), jnp.float32),
            pltpu.VMEM((H, NCOL), jnp.float32),
            pltpu.VMEM((LC, LCH, NCOL), jnp.float32),
            pltpu.SemaphoreType.DMA((K,)),
            pltpu.SemaphoreType.DMA((K,)),
            pltpu.SemaphoreType.DMA((K,)),
            pltpu.SemaphoreType.DMA((K,)),
            pltpu.SemaphoreType.DMA((K,)),
            pltpu.SemaphoreType.DMA((K,)),
            pltpu.SemaphoreType.DMA((K,)),
            pltpu.SemaphoreType.DMA((LC,)),
            pltpu.SemaphoreType.DMA((LC,)),
        ],
        compiler_params=pltpu.CompilerParams(
            collective_id=0, vmem_limit_bytes=64 * 1024 * 1024,
        ),
    )(x)


# device time: 127675 ns/iter; 1.0435x vs baseline; 1.0435x over previous
import jax
import jax.numpy as jnp
from jax import lax
from jax.experimental import pallas as pl
from jax.experimental.pallas import tpu as pltpu

K = 16
LC = 8


def kernel(x):
    M, N = x.shape
    NCOL = N // 2
    H = M // 2
    M_OUT = 2 * M
    CH = H // K
    LCH = M // LC

    def body(x_ref, out_ref, vsend, vrecv1, vrecv2, vstage,
             stage_sems, p1_send, p1_recv, p2_send, p2_recv,
             st1_sems, st2_sems, lc_in_sems, lc_out_sems):
        mx = lax.axis_index("x")
        my = lax.axis_index("y")

        barrier = pltpu.get_barrier_semaphore()
        pl.semaphore_signal(barrier, inc=1, device_id=(1 - mx, my),
                            device_id_type=pl.DeviceIdType.MESH)
        pl.semaphore_signal(barrier, inc=1, device_id=(mx, 1 - my),
                            device_id_type=pl.DeviceIdType.MESH)
        pl.semaphore_wait(barrier, 2)

        stage = []
        for k in range(K):
            c = pltpu.make_async_copy(
                x_ref.at[pl.ds(my * H + k * CH, CH),
                         pl.ds((1 - mx) * NCOL, NCOL)],
                vsend.at[pl.ds(k * CH, CH), :],
                stage_sems.at[k],
            )
            c.start()
            stage.append(c)
        p1 = []
        for k in range(K):
            stage[k].wait()
            r = pltpu.make_async_remote_copy(
                src_ref=vsend.at[pl.ds(k * CH, CH), :],
                dst_ref=vrecv1.at[pl.ds(k * CH, CH), :],
                send_sem=p1_send.at[k],
                recv_sem=p1_recv.at[k],
                device_id=(1 - mx, my),
                device_id_type=pl.DeviceIdType.MESH,
            )
            r.start()
            p1.append(r)

        lc_in = []
        for k in range(LC):
            c = pltpu.make_async_copy(
                x_ref.at[pl.ds(k * LCH, LCH), pl.ds(mx * NCOL, NCOL)],
                vstage.at[k],
                lc_in_sems.at[k],
            )
            c.start()
            lc_in.append(c)

        p2 = []
        st1 = []
        lc_out = []
        for k in range(K):
            p1[k].wait_recv()
            r = pltpu.make_async_remote_copy(
                src_ref=vrecv1.at[pl.ds(k * CH, CH), :],
                dst_ref=vrecv2.at[pl.ds(k * CH, CH), :],
                send_sem=p2_send.at[k],
                recv_sem=p2_recv.at[k],
                device_id=(mx, 1 - my),
                device_id_type=pl.DeviceIdType.MESH,
            )
            r.start()
            p2.append(r)
            s = pltpu.make_async_copy(
                vrecv1.at[pl.ds(k * CH, CH), :],
                out_ref.at[pl.ds((1 - mx) * M + my * H + k * CH, CH), :],
                st1_sems.at[k],
            )
            s.start()
            st1.append(s)
            if k < LC:
                lc_in[k].wait()
                c = pltpu.make_async_copy(
                    vstage.at[k],
                    out_ref.at[pl.ds(mx * M + k * LCH, LCH), :],
                    lc_out_sems.at[k],
                )
                c.start()
                lc_out.append(c)

        st2 = []
        for k in range(K):
            p2[k].wait_recv()
            s = pltpu.make_async_copy(
                vrecv2.at[pl.ds(k * CH, CH), :],
                out_ref.at[pl.ds((1 - mx) * M + (1 - my) * H + k * CH, CH), :],
                st2_sems.at[k],
            )
            s.start()
            st2.append(s)

        for k in range(K):
            p1[k].wait_send()
            p2[k].wait_send()
            st1[k].wait()
            st2[k].wait()
        for k in range(LC):
            lc_out[k].wait()

    return pl.pallas_call(
        body,
        out_shape=jax.ShapeDtypeStruct((M_OUT, NCOL), jnp.float32),
        in_specs=[pl.BlockSpec(memory_space=pl.ANY)],
        out_specs=pl.BlockSpec(memory_space=pl.ANY),
        scratch_shapes=[
            pltpu.VMEM((H, NCOL), jnp.float32),
            pltpu.VMEM((H, NCOL), jnp.float32),
            pltpu.VMEM((H, NCOL), jnp.float32),
            pltpu.VMEM((LC, LCH, NCOL), jnp.float32),
            pltpu.SemaphoreType.DMA((K,)),
            pltpu.SemaphoreType.DMA((K,)),
            pltpu.SemaphoreType.DMA((K,)),
            pltpu.SemaphoreType.DMA((K,)),
            pltpu.SemaphoreType.DMA((K,)),
            pltpu.SemaphoreType.DMA((K,)),
            pltpu.SemaphoreType.DMA((K,)),
            pltpu.SemaphoreType.DMA((LC,)),
            pltpu.SemaphoreType.DMA((LC,)),
        ],
        compiler_params=pltpu.CompilerParams(
            collective_id=0, vmem_limit_bytes=64 * 1024 * 1024,
        ),
    )(x)
